# baseline (speedup 1.0000x reference)
import functools

import jax
import jax.numpy as jnp
from jax import lax
from jax.experimental import pallas as pl
from jax.experimental.pallas import tpu as pltpu
from jax.experimental.pallas import tpu_sc as plsc

DM = 1024
DS = 32768
KTOP = 32

NEGINF = float("-inf")
IBIG = 2**30


NB_ENC = 32
NBLK = DS // NB_ENC
FB = 128
NFB = DS // FB
FB_PER_STEP = NBLK // FB


def _enc_body(x_ref, we_ref, be_ref, o_ref, bmax_ref, tau_ref):
    i = pl.program_id(0)
    acc = lax.dot_general(
        x_ref[...], we_ref[...],
        dimension_numbers=(((1,), (1,)), ((), ())),
        preferred_element_type=jnp.float32,
    ) + be_ref[0]
    o_ref[...] = acc
    bm = jnp.max(acc.reshape(128, FB_PER_STEP, FB), axis=2)
    for k in range(NB_ENC):
        @pl.when(i == k)
        def _():
            bmax_ref[:, k * FB_PER_STEP:(k + 1) * FB_PER_STEP] = bm

    @pl.when(i == NB_ENC - 1)
    def _():
        bmx = bmax_ref[...]
        iota = lax.broadcasted_iota(jnp.int32, (128, NFB), 1)

        def step(_, carry):
            w, _ = carry
            m = jnp.max(w, axis=1, keepdims=True)
            first = jnp.min(jnp.where(w == m, iota, IBIG),
                            axis=1, keepdims=True)
            w = jnp.where(iota == first, NEGINF, w)
            return (w, m)

        _, tau = lax.fori_loop(0, KTOP, step,
                               (bmx, jnp.zeros((128, 1), jnp.float32)))
        bmax_ref[...] = bmx
        tau_ref[...] = jnp.broadcast_to(tau, (128, 128))


def _encoder(x, We, be):
    be3 = be.reshape(NB_ENC, 1, NBLK)
    return pl.pallas_call(
        _enc_body,
        grid=(NB_ENC,),
        in_specs=[
            pl.BlockSpec((128, DM), lambda i: (0, 0)),
            pl.BlockSpec((NBLK, DM), lambda i: (i, 0)),
            pl.BlockSpec((1, 1, NBLK), lambda i: (i, 0, 0)),
        ],
        out_specs=[
            pl.BlockSpec((128, NBLK), lambda i: (0, i)),
            pl.BlockSpec((128, NFB), lambda i: (0, 0)),
            pl.BlockSpec((128, 128), lambda i: (0, 0)),
        ],
        out_shape=[
            jax.ShapeDtypeStruct((128, DS), jnp.float32),
            jax.ShapeDtypeStruct((128, NFB), jnp.float32),
            jax.ShapeDtypeStruct((128, 128), jnp.float32),
        ],
    )(x, We, be3)


BLKV = FB // 16
CMAX = 2048
CPAD = CMAX + 16
NW = 32
RPW = 128 // NW


def _select32(read_vreg, nv, lastv, lastp):
    lv = jnp.full((16,), lastv, jnp.float32)
    lp = jnp.full((16,), lastp, jnp.int32)

    def p1(i, carry):
        bestv, bestp = carry
        v, p, ok = read_vreg(i)
        elig = ok & ((v < lv) | ((v == lv) & (p > lp)))
        ve = jnp.where(elig, v, NEGINF)
        upd = (ve > bestv) | ((ve == bestv) & (p < bestp))
        return (jnp.where(upd, ve, bestv), jnp.where(upd, p, bestp))

    bestv, bestp = lax.fori_loop(
        0, nv, p1,
        (jnp.full((16,), NEGINF, jnp.float32),
         jnp.full((16,), IBIG, jnp.int32)))
    vstar = jnp.max(bestv)
    vs = jnp.full((16,), vstar, jnp.float32)
    pstar = jnp.min(jnp.where(bestv == vs, bestp, IBIG))
    return vstar, pstar


def _sc_topk_latents(pre, bmax, tau):
    mesh = plsc.VectorSubcoreMesh(core_axis_name="c", subcore_axis_name="s",
                                  num_cores=2, num_subcores=16)

    @functools.partial(
        pl.kernel,
        out_type=jax.ShapeDtypeStruct((128, DS), jnp.float32),
        mesh=mesh,
        compiler_params=pltpu.CompilerParams(needs_layout_passes=False),
        scratch_types=[
            pltpu.VMEM((3 * DS,), jnp.float32),
            pltpu.VMEM((CPAD,), jnp.int32),
            pltpu.VMEM((RPW, NFB), jnp.float32),
            pltpu.VMEM((RPW, 128), jnp.float32),
            pltpu.VMEM((NFB + 16,), jnp.int32),
            pltpu.VMEM((KTOP,), jnp.float32),
            pltpu.VMEM((KTOP,), jnp.int32),
            pltpu.SemaphoreType.DMA,
            pltpu.SemaphoreType.DMA,
        ],
    )
    def body(pre_h, bmax_h, tau_h, lat_h, rowb, candi,
             bmaxv, tauv, hotv, selv, seli, insem, outsem):
        cid = lax.axis_index("c")
        sid = lax.axis_index("s")
        wid = sid * 2 + cid
        base = wid * RPW
        iota = lax.iota(jnp.int32, 16)
        zero16 = jnp.zeros((16,), jnp.float32)

        pltpu.sync_copy(bmax_h.at[pl.ds(base, RPW)], bmaxv)
        pltpu.sync_copy(tau_h.at[pl.ds(base, RPW)], tauv)

        def cinit(i, c):
            candi[pl.ds(i * 16, 16)] = jnp.zeros((16,), jnp.int32)
            return c
        lax.fori_loop(0, CPAD // 16, cinit, 0)

        pltpu.make_async_copy(pre_h.at[base], rowb.at[pl.ds(0, DS)],
                              insem).start()
        pltpu.make_async_copy(pre_h.at[base + 1], rowb.at[pl.ds(DS, DS)],
                              insem).start()

        def row_body(j, c):
            slot = lax.rem(j, 3)
            soff = slot * DS
            nslot = lax.rem(j + 2, 3)
            row = base + j
            pltpu.make_async_copy(pre_h.at[row], rowb.at[pl.ds(soff, DS)],
                                  insem).wait()

            @pl.when(j > 0)
            def _():
                pltpu.make_async_copy(rowb.at[pl.ds(nslot * DS, DS)],
                                      lat_h.at[row - 1], outsem).wait()

            @pl.when(j + 2 < RPW)
            def _():
                pltpu.make_async_copy(pre_h.at[row + 2],
                                      rowb.at[pl.ds(nslot * DS, DS)],
                                      insem).start()

            tvec = tauv[j, pl.ds(0, 16)]

            def hot_body(jj, cnt):
                bv = bmaxv[j, pl.ds(jj * 16, 16)]
                m = bv >= tvec
                offs = cnt + plsc.cumsum(m.astype(jnp.int32)) - 1
                plsc.store_scatter(hotv, [offs], jj * 16 + iota, mask=m)
                return cnt + plsc.all_reduce_population_count(m)

            nhot_v = lax.fori_loop(0, NFB // 16, hot_body,
                                   jnp.zeros((16,), jnp.int32))
            nhot = jnp.max(nhot_v)

            def blk_body(i, cnt):
                b = hotv[pl.ds(i, 16)][0]
                for u in range(BLKV):
                    v = rowb[pl.ds(soff + (b * BLKV + u) * 16, 16)]
                    m = v >= tvec
                    offs = cnt + plsc.cumsum(m.astype(jnp.int32)) - 1
                    mm = m & (offs < CMAX)
                    plsc.store_scatter(candi, [offs], b * FB + u * 16 + iota,
                                       mask=mm)
                    cnt = cnt + plsc.all_reduce_population_count(m)
                return cnt

            c_v = lax.fori_loop(0, nhot, blk_body,
                                jnp.zeros((16,), jnp.int32))
            c_true = jnp.max(c_v)
            cs = jnp.full((16,), jnp.minimum(c_true, jnp.int32(CMAX)),
                          jnp.int32)
            soff_v = jnp.full((16,), soff, jnp.int32)
            true16 = iota >= 0

            def run_select(rd, nv):
                def t_body(t, carry):
                    lastv, lastp, sv0, sv1, si0, si1 = carry
                    vstar, pstar = _select32(rd, nv, lastv, lastp)
                    m0 = iota == t
                    m1 = iota == (t - 16)
                    sv0 = jnp.where(m0, vstar, sv0)
                    sv1 = jnp.where(m1, vstar, sv1)
                    si0 = jnp.where(m0, pstar, si0)
                    si1 = jnp.where(m1, pstar, si1)
                    return (vstar, pstar, sv0, sv1, si0, si1)

                izero = jnp.zeros((16,), jnp.int32)
                out = lax.fori_loop(
                    0, KTOP, t_body,
                    (jnp.float32(jnp.inf), jnp.int32(-1),
                     zero16, zero16, izero, izero))
                selv[pl.ds(0, 16)] = out[2]
                selv[pl.ds(16, 16)] = out[3]
                seli[pl.ds(0, 16)] = out[4]
                seli[pl.ds(16, 16)] = out[5]
                return 0

            def normal(_):
                nv = (jnp.minimum(c_true, jnp.int32(CMAX)) + 15) // 16

                def rd(i):
                    ci = candi[pl.ds(i * 16, 16)]
                    v = plsc.load_gather(rowb, [soff_v + ci])
                    return v, ci, (i * 16 + iota) < cs

                return run_select(rd, nv)

            def fallback(_):
                def rd(i):
                    return (rowb[pl.ds(soff + i * 16, 16)], i * 16 + iota,
                            true16)

                return run_select(rd, DS // 16)

            lax.cond(c_true <= CMAX, normal, fallback, 0)

            def zb(i, c2):
                for u in range(8):
                    rowb[pl.ds(soff + i * 128 + u * 16, 16)] = zero16
                return c2
            lax.fori_loop(0, DS // 128, zb, 0)
            for u in range(KTOP // 16):
                ii = seli[pl.ds(u * 16, 16)]
                vv = selv[pl.ds(u * 16, 16)]
                plsc.store_scatter(rowb, [soff_v + ii], vv)
            pltpu.make_async_copy(rowb.at[pl.ds(soff, DS)], lat_h.at[row],
                                  outsem).start()
            return c

        lax.fori_loop(0, RPW, row_body, 0)
        pltpu.make_async_copy(rowb.at[pl.ds(lax.rem(RPW - 1, 3) * DS, DS)],
                              lat_h.at[base + RPW - 1], outsem).wait()

    return body(pre, bmax, tau)


NB_DEC = 16
KBLK = DS // NB_DEC


def _dec_body(lat_ref, wd_ref, o_ref):
    k = pl.program_id(0)

    @pl.when(k == 0)
    def _():
        o_ref[...] = jnp.zeros_like(o_ref)

    o_ref[...] += lax.dot_general(
        lat_ref[...], wd_ref[...],
        dimension_numbers=(((1,), (1,)), ((), ())),
        preferred_element_type=jnp.float32,
    )


def _decoder(latents, Wd):
    return pl.pallas_call(
        _dec_body,
        grid=(NB_DEC,),
        in_specs=[
            pl.BlockSpec((128, KBLK), lambda k: (0, k)),
            pl.BlockSpec((DM, KBLK), lambda k: (0, k)),
        ],
        out_specs=pl.BlockSpec((128, DM), lambda k: (0, 0)),
        out_shape=jax.ShapeDtypeStruct((128, DM), jnp.float32),
    )(latents, Wd)


def kernel(x, We, be, Wd):
    pre, bmax, tau = _encoder(x, We, be)
    latents = _sc_topk_latents(pre, bmax, tau)
    recon = _decoder(latents, Wd)
    return (recon, latents, pre)


# speedup vs baseline: 1.0845x; 1.0845x over previous
import functools

import jax
import jax.numpy as jnp
from jax import lax
from jax.experimental import pallas as pl
from jax.experimental.pallas import tpu as pltpu
from jax.experimental.pallas import tpu_sc as plsc

DM = 1024
DS = 32768
KTOP = 32

NEGINF = float("-inf")
IBIG = 2**30


NB_ENC = 16
NBLK = DS // NB_ENC
FB = 128
NFB = DS // FB
FB_PER_STEP = NBLK // FB


def _enc_body(x_ref, we_ref, be_ref, o_ref, bmax_ref, tau_ref):
    i = pl.program_id(0)
    acc = lax.dot_general(
        x_ref[...], we_ref[...],
        dimension_numbers=(((1,), (1,)), ((), ())),
        preferred_element_type=jnp.float32,
    ) + be_ref[0]
    o_ref[...] = acc
    bm = jnp.max(acc.reshape(128, FB_PER_STEP, FB), axis=2)
    for k in range(NB_ENC):
        @pl.when(i == k)
        def _():
            bmax_ref[:, k * FB_PER_STEP:(k + 1) * FB_PER_STEP] = bm

    @pl.when(i == NB_ENC - 1)
    def _():
        bmx = bmax_ref[...]
        iota = lax.broadcasted_iota(jnp.int32, (128, NFB), 1)

        def step(_, carry):
            w, _ = carry
            m = jnp.max(w, axis=1, keepdims=True)
            first = jnp.min(jnp.where(w == m, iota, IBIG),
                            axis=1, keepdims=True)
            w = jnp.where(iota == first, NEGINF, w)
            return (w, m)

        _, tau = lax.fori_loop(0, KTOP, step,
                               (bmx, jnp.zeros((128, 1), jnp.float32)))
        bmax_ref[...] = bmx
        tau_ref[...] = jnp.broadcast_to(tau, (128, 128))


def _encoder(x, We, be):
    be3 = be.reshape(NB_ENC, 1, NBLK)
    return pl.pallas_call(
        _enc_body,
        grid=(NB_ENC,),
        in_specs=[
            pl.BlockSpec((128, DM), lambda i: (0, 0)),
            pl.BlockSpec((NBLK, DM), lambda i: (i, 0)),
            pl.BlockSpec((1, 1, NBLK), lambda i: (i, 0, 0)),
        ],
        out_specs=[
            pl.BlockSpec((128, NBLK), lambda i: (0, i)),
            pl.BlockSpec((128, NFB), lambda i: (0, 0)),
            pl.BlockSpec((128, 128), lambda i: (0, 0)),
        ],
        out_shape=[
            jax.ShapeDtypeStruct((128, DS), jnp.float32),
            jax.ShapeDtypeStruct((128, NFB), jnp.float32),
            jax.ShapeDtypeStruct((128, 128), jnp.float32),
        ],
    )(x, We, be3)


BLKV = FB // 16
CMAX = 2048
CPAD = CMAX + 16
NW = 32
RPW = 128 // NW


def _select32(read_vreg, nv, lastv, lastp):
    lv = jnp.full((16,), lastv, jnp.float32)
    lp = jnp.full((16,), lastp, jnp.int32)

    def p1(i, carry):
        bestv, bestp = carry
        v, p, ok = read_vreg(i)
        elig = ok & ((v < lv) | ((v == lv) & (p > lp)))
        ve = jnp.where(elig, v, NEGINF)
        upd = (ve > bestv) | ((ve == bestv) & (p < bestp))
        return (jnp.where(upd, ve, bestv), jnp.where(upd, p, bestp))

    bestv, bestp = lax.fori_loop(
        0, nv, p1,
        (jnp.full((16,), NEGINF, jnp.float32),
         jnp.full((16,), IBIG, jnp.int32)))
    vstar = jnp.max(bestv)
    vs = jnp.full((16,), vstar, jnp.float32)
    pstar = jnp.min(jnp.where(bestv == vs, bestp, IBIG))
    return vstar, pstar


def _sc_topk_latents(pre, bmax, tau):
    mesh = plsc.VectorSubcoreMesh(core_axis_name="c", subcore_axis_name="s",
                                  num_cores=2, num_subcores=16)

    @functools.partial(
        pl.kernel,
        out_type=jax.ShapeDtypeStruct((128, DS), jnp.float32),
        mesh=mesh,
        compiler_params=pltpu.CompilerParams(needs_layout_passes=False),
        scratch_types=[
            pltpu.VMEM((2, DS), jnp.float32),
            pltpu.VMEM((DS,), jnp.float32),
            pltpu.VMEM((CPAD,), jnp.int32),
            pltpu.VMEM((RPW, NFB), jnp.float32),
            pltpu.VMEM((RPW, 128), jnp.float32),
            pltpu.VMEM((NFB + 16,), jnp.int32),
            pltpu.VMEM((2, KTOP), jnp.float32),
            pltpu.VMEM((2, KTOP), jnp.int32),
            pltpu.SemaphoreType.DMA,
            pltpu.SemaphoreType.DMA,
        ],
    )
    def body(pre_h, bmax_h, tau_h, lat_h, rowb, latb, candi,
             bmaxv, tauv, hotv, selv, seli, insem, outsem):
        cid = lax.axis_index("c")
        sid = lax.axis_index("s")
        wid = sid * 2 + cid
        base = wid * RPW
        iota = lax.iota(jnp.int32, 16)
        zero16 = jnp.zeros((16,), jnp.float32)

        pltpu.sync_copy(bmax_h.at[pl.ds(base, RPW)], bmaxv)
        pltpu.sync_copy(tau_h.at[pl.ds(base, RPW)], tauv)

        def zbody(i, c):
            for u in range(8):
                latb[pl.ds(i * 128 + u * 16, 16)] = zero16
            return c
        lax.fori_loop(0, DS // 128, zbody, 0)

        def cinit(i, c):
            candi[pl.ds(i * 16, 16)] = jnp.zeros((16,), jnp.int32)
            return c
        lax.fori_loop(0, CPAD // 16, cinit, 0)

        pltpu.make_async_copy(pre_h.at[base], rowb.at[0], insem).start()
        pltpu.make_async_copy(pre_h.at[base + 1], rowb.at[1], insem).start()

        def row_body(j, c):
            slot = lax.rem(j, 2)
            row = base + j
            pltpu.make_async_copy(pre_h.at[row], rowb.at[slot], insem).wait()

            tvec = tauv[j, pl.ds(0, 16)]

            def hot_body(jj, cnt):
                bv = bmaxv[j, pl.ds(jj * 16, 16)]
                m = bv >= tvec
                offs = cnt + plsc.cumsum(m.astype(jnp.int32)) - 1
                plsc.store_scatter(hotv, [offs], jj * 16 + iota, mask=m)
                return cnt + plsc.all_reduce_population_count(m)

            nhot_v = lax.fori_loop(0, NFB // 16, hot_body,
                                   jnp.zeros((16,), jnp.int32))
            nhot = jnp.max(nhot_v)

            def blk_body(i, cnt):
                b = hotv[pl.ds(i, 16)][0]
                for u in range(BLKV):
                    v = rowb[slot, pl.ds((b * BLKV + u) * 16, 16)]
                    m = v >= tvec
                    offs = cnt + plsc.cumsum(m.astype(jnp.int32)) - 1
                    mm = m & (offs < CMAX)
                    plsc.store_scatter(candi, [offs], b * FB + u * 16 + iota,
                                       mask=mm)
                    cnt = cnt + plsc.all_reduce_population_count(m)
                return cnt

            c_v = lax.fori_loop(0, nhot, blk_body,
                                jnp.zeros((16,), jnp.int32))
            c_true = jnp.max(c_v)
            cs = jnp.full((16,), jnp.minimum(c_true, jnp.int32(CMAX)),
                          jnp.int32)
            slot_v = jnp.full((16,), slot, jnp.int32)
            true16 = iota >= 0

            def run_select(rd, nv):
                def t_body(t, carry):
                    lastv, lastp, sv0, sv1, si0, si1 = carry
                    vstar, pstar = _select32(rd, nv, lastv, lastp)
                    m0 = iota == t
                    m1 = iota == (t - 16)
                    sv0 = jnp.where(m0, vstar, sv0)
                    sv1 = jnp.where(m1, vstar, sv1)
                    si0 = jnp.where(m0, pstar, si0)
                    si1 = jnp.where(m1, pstar, si1)
                    return (vstar, pstar, sv0, sv1, si0, si1)

                izero = jnp.zeros((16,), jnp.int32)
                out = lax.fori_loop(
                    0, KTOP, t_body,
                    (jnp.float32(jnp.inf), jnp.int32(-1),
                     zero16, zero16, izero, izero))
                selv[slot, pl.ds(0, 16)] = out[2]
                selv[slot, pl.ds(16, 16)] = out[3]
                seli[slot, pl.ds(0, 16)] = out[4]
                seli[slot, pl.ds(16, 16)] = out[5]
                return 0

            def normal(_):
                nv = (jnp.minimum(c_true, jnp.int32(CMAX)) + 15) // 16

                def rd(i):
                    ci = candi[pl.ds(i * 16, 16)]
                    v = plsc.load_gather(rowb, [slot_v, ci])
                    return v, ci, (i * 16 + iota) < cs

                return run_select(rd, nv)

            def fallback(_):
                def rd(i):
                    return (rowb[slot, pl.ds(i * 16, 16)], i * 16 + iota,
                            true16)

                return run_select(rd, DS // 16)

            lax.cond(c_true <= CMAX, normal, fallback, 0)

            @pl.when(j + 2 < RPW)
            def _():
                pltpu.make_async_copy(pre_h.at[row + 2], rowb.at[slot],
                                      insem).start()

            @pl.when(j > 0)
            def _():
                pltpu.make_async_copy(latb, lat_h.at[row - 1], outsem).wait()
                ps = lax.rem(j + 1, 2)
                for u in range(KTOP // 16):
                    ii = seli[ps, pl.ds(u * 16, 16)]
                    plsc.store_scatter(latb, [ii], zero16)

            for u in range(KTOP // 16):
                ii = seli[slot, pl.ds(u * 16, 16)]
                vv = selv[slot, pl.ds(u * 16, 16)]
                plsc.store_scatter(latb, [ii], vv)
            pltpu.make_async_copy(latb, lat_h.at[row], outsem).start()
            return c

        lax.fori_loop(0, RPW, row_body, 0)
        pltpu.make_async_copy(latb, lat_h.at[base + RPW - 1], outsem).wait()

    return body(pre, bmax, tau)


NB_DEC = 16
KBLK = DS // NB_DEC


def _dec_body(lat_ref, wd_ref, o_ref):
    k = pl.program_id(0)

    @pl.when(k == 0)
    def _():
        o_ref[...] = jnp.zeros_like(o_ref)

    o_ref[...] += lax.dot_general(
        lat_ref[...], wd_ref[...],
        dimension_numbers=(((1,), (1,)), ((), ())),
        preferred_element_type=jnp.float32,
    )


def _decoder(latents, Wd):
    return pl.pallas_call(
        _dec_body,
        grid=(NB_DEC,),
        in_specs=[
            pl.BlockSpec((128, KBLK), lambda k: (0, k)),
            pl.BlockSpec((DM, KBLK), lambda k: (0, k)),
        ],
        out_specs=pl.BlockSpec((128, DM), lambda k: (0, 0)),
        out_shape=jax.ShapeDtypeStruct((128, DM), jnp.float32),
    )(latents, Wd)


def kernel(x, We, be, Wd):
    pre, bmax, tau = _encoder(x, We, be)
    latents = _sc_topk_latents(pre, bmax, tau)
    recon = _decoder(latents, Wd)
    return (recon, latents, pre)


# speedup vs baseline: 1.0916x; 1.0065x over previous
import functools

import jax
import jax.numpy as jnp
from jax import lax
from jax.experimental import pallas as pl
from jax.experimental.pallas import tpu as pltpu
from jax.experimental.pallas import tpu_sc as plsc

DM = 1024
DS = 32768
KTOP = 32

NEGINF = float("-inf")
IBIG = 2**30


NB_ENC = 8
NBLK = DS // NB_ENC
FB = 128
NFB = DS // FB
FB_PER_STEP = NBLK // FB


def _enc_body(x_ref, we_ref, be_ref, o_ref, bmax_ref, tau_ref):
    i = pl.program_id(0)
    acc = lax.dot_general(
        x_ref[...], we_ref[...],
        dimension_numbers=(((1,), (1,)), ((), ())),
        preferred_element_type=jnp.float32,
    ) + be_ref[0]
    o_ref[...] = acc
    bm = jnp.max(acc.reshape(128, FB_PER_STEP, FB), axis=2)
    for k in range(NB_ENC):
        @pl.when(i == k)
        def _():
            bmax_ref[:, k * FB_PER_STEP:(k + 1) * FB_PER_STEP] = bm

    @pl.when(i == NB_ENC - 1)
    def _():
        bmx = bmax_ref[...]
        iota = lax.broadcasted_iota(jnp.int32, (128, NFB), 1)

        def step(_, carry):
            w, _ = carry
            m = jnp.max(w, axis=1, keepdims=True)
            first = jnp.min(jnp.where(w == m, iota, IBIG),
                            axis=1, keepdims=True)
            w = jnp.where(iota == first, NEGINF, w)
            return (w, m)

        _, tau = lax.fori_loop(0, KTOP, step,
                               (bmx, jnp.zeros((128, 1), jnp.float32)))
        bmax_ref[...] = bmx
        tau_ref[...] = jnp.broadcast_to(tau, (128, 128))


def _encoder(x, We, be):
    be3 = be.reshape(NB_ENC, 1, NBLK)
    return pl.pallas_call(
        _enc_body,
        grid=(NB_ENC,),
        in_specs=[
            pl.BlockSpec((128, DM), lambda i: (0, 0)),
            pl.BlockSpec((NBLK, DM), lambda i: (i, 0)),
            pl.BlockSpec((1, 1, NBLK), lambda i: (i, 0, 0)),
        ],
        out_specs=[
            pl.BlockSpec((128, NBLK), lambda i: (0, i)),
            pl.BlockSpec((128, NFB), lambda i: (0, 0)),
            pl.BlockSpec((128, 128), lambda i: (0, 0)),
        ],
        out_shape=[
            jax.ShapeDtypeStruct((128, DS), jnp.float32),
            jax.ShapeDtypeStruct((128, NFB), jnp.float32),
            jax.ShapeDtypeStruct((128, 128), jnp.float32),
        ],
    )(x, We, be3)


BLKV = FB // 16
CMAX = 2048
CPAD = CMAX + 16
NW = 32
RPW = 128 // NW


def _select32(read_vreg, nv, lastv, lastp):
    lv = jnp.full((16,), lastv, jnp.float32)
    lp = jnp.full((16,), lastp, jnp.int32)

    def p1(i, carry):
        bestv, bestp = carry
        v, p, ok = read_vreg(i)
        elig = ok & ((v < lv) | ((v == lv) & (p > lp)))
        ve = jnp.where(elig, v, NEGINF)
        upd = (ve > bestv) | ((ve == bestv) & (p < bestp))
        return (jnp.where(upd, ve, bestv), jnp.where(upd, p, bestp))

    bestv, bestp = lax.fori_loop(
        0, nv, p1,
        (jnp.full((16,), NEGINF, jnp.float32),
         jnp.full((16,), IBIG, jnp.int32)))
    vstar = jnp.max(bestv)
    vs = jnp.full((16,), vstar, jnp.float32)
    pstar = jnp.min(jnp.where(bestv == vs, bestp, IBIG))
    return vstar, pstar


def _sc_topk_latents(pre, bmax, tau):
    mesh = plsc.VectorSubcoreMesh(core_axis_name="c", subcore_axis_name="s",
                                  num_cores=2, num_subcores=16)

    @functools.partial(
        pl.kernel,
        out_type=jax.ShapeDtypeStruct((128, DS), jnp.float32),
        mesh=mesh,
        compiler_params=pltpu.CompilerParams(needs_layout_passes=False),
        scratch_types=[
            pltpu.VMEM((2, DS), jnp.float32),
            pltpu.VMEM((DS,), jnp.float32),
            pltpu.VMEM((CPAD,), jnp.int32),
            pltpu.VMEM((RPW, NFB), jnp.float32),
            pltpu.VMEM((RPW, 128), jnp.float32),
            pltpu.VMEM((NFB + 16,), jnp.int32),
            pltpu.VMEM((2, KTOP), jnp.float32),
            pltpu.VMEM((2, KTOP), jnp.int32),
            pltpu.SemaphoreType.DMA,
            pltpu.SemaphoreType.DMA,
        ],
    )
    def body(pre_h, bmax_h, tau_h, lat_h, rowb, latb, candi,
             bmaxv, tauv, hotv, selv, seli, insem, outsem):
        cid = lax.axis_index("c")
        sid = lax.axis_index("s")
        wid = sid * 2 + cid
        base = wid * RPW
        iota = lax.iota(jnp.int32, 16)
        zero16 = jnp.zeros((16,), jnp.float32)

        pltpu.sync_copy(bmax_h.at[pl.ds(base, RPW)], bmaxv)
        pltpu.sync_copy(tau_h.at[pl.ds(base, RPW)], tauv)

        def zbody(i, c):
            for u in range(8):
                latb[pl.ds(i * 128 + u * 16, 16)] = zero16
            return c
        lax.fori_loop(0, DS // 128, zbody, 0)

        def cinit(i, c):
            candi[pl.ds(i * 16, 16)] = jnp.zeros((16,), jnp.int32)
            return c
        lax.fori_loop(0, CPAD // 16, cinit, 0)

        pltpu.make_async_copy(pre_h.at[base], rowb.at[0], insem).start()
        pltpu.make_async_copy(pre_h.at[base + 1], rowb.at[1], insem).start()

        def row_body(j, c):
            slot = lax.rem(j, 2)
            row = base + j
            pltpu.make_async_copy(pre_h.at[row], rowb.at[slot], insem).wait()

            tvec = tauv[j, pl.ds(0, 16)]

            def hot_body(jj, cnt):
                bv = bmaxv[j, pl.ds(jj * 16, 16)]
                m = bv >= tvec
                offs = cnt + plsc.cumsum(m.astype(jnp.int32)) - 1
                plsc.store_scatter(hotv, [offs], jj * 16 + iota, mask=m)
                return cnt + plsc.all_reduce_population_count(m)

            nhot_v = lax.fori_loop(0, NFB // 16, hot_body,
                                   jnp.zeros((16,), jnp.int32))
            nhot = jnp.max(nhot_v)

            def blk_body(i, cnt):
                b = hotv[pl.ds(i, 16)][0]
                for u in range(BLKV):
                    v = rowb[slot, pl.ds((b * BLKV + u) * 16, 16)]
                    m = v >= tvec
                    offs = cnt + plsc.cumsum(m.astype(jnp.int32)) - 1
                    mm = m & (offs < CMAX)
                    plsc.store_scatter(candi, [offs], b * FB + u * 16 + iota,
                                       mask=mm)
                    cnt = cnt + plsc.all_reduce_population_count(m)
                return cnt

            c_v = lax.fori_loop(0, nhot, blk_body,
                                jnp.zeros((16,), jnp.int32))
            c_true = jnp.max(c_v)
            cs = jnp.full((16,), jnp.minimum(c_true, jnp.int32(CMAX)),
                          jnp.int32)
            slot_v = jnp.full((16,), slot, jnp.int32)
            true16 = iota >= 0

            def run_select(rd, nv):
                def t_body(t, carry):
                    lastv, lastp, sv0, sv1, si0, si1 = carry
                    vstar, pstar = _select32(rd, nv, lastv, lastp)
                    m0 = iota == t
                    m1 = iota == (t - 16)
                    sv0 = jnp.where(m0, vstar, sv0)
                    sv1 = jnp.where(m1, vstar, sv1)
                    si0 = jnp.where(m0, pstar, si0)
                    si1 = jnp.where(m1, pstar, si1)
                    return (vstar, pstar, sv0, sv1, si0, si1)

                izero = jnp.zeros((16,), jnp.int32)
                out = lax.fori_loop(
                    0, KTOP, t_body,
                    (jnp.float32(jnp.inf), jnp.int32(-1),
                     zero16, zero16, izero, izero))
                selv[slot, pl.ds(0, 16)] = out[2]
                selv[slot, pl.ds(16, 16)] = out[3]
                seli[slot, pl.ds(0, 16)] = out[4]
                seli[slot, pl.ds(16, 16)] = out[5]
                return 0

            def normal(_):
                nv = (jnp.minimum(c_true, jnp.int32(CMAX)) + 15) // 16

                def rd(i):
                    ci = candi[pl.ds(i * 16, 16)]
                    v = plsc.load_gather(rowb, [slot_v, ci])
                    return v, ci, (i * 16 + iota) < cs

                return run_select(rd, nv)

            def fallback(_):
                def rd(i):
                    return (rowb[slot, pl.ds(i * 16, 16)], i * 16 + iota,
                            true16)

                return run_select(rd, DS // 16)

            lax.cond(c_true <= CMAX, normal, fallback, 0)

            @pl.when(j + 2 < RPW)
            def _():
                pltpu.make_async_copy(pre_h.at[row + 2], rowb.at[slot],
                                      insem).start()

            @pl.when(j > 0)
            def _():
                pltpu.make_async_copy(latb, lat_h.at[row - 1], outsem).wait()
                ps = lax.rem(j + 1, 2)
                for u in range(KTOP // 16):
                    ii = seli[ps, pl.ds(u * 16, 16)]
                    plsc.store_scatter(latb, [ii], zero16)

            for u in range(KTOP // 16):
                ii = seli[slot, pl.ds(u * 16, 16)]
                vv = selv[slot, pl.ds(u * 16, 16)]
                plsc.store_scatter(latb, [ii], vv)
            pltpu.make_async_copy(latb, lat_h.at[row], outsem).start()
            return c

        lax.fori_loop(0, RPW, row_body, 0)
        pltpu.make_async_copy(latb, lat_h.at[base + RPW - 1], outsem).wait()

    return body(pre, bmax, tau)


NB_DEC = 8
KBLK = DS // NB_DEC


def _dec_body(lat_ref, wd_ref, o_ref):
    k = pl.program_id(0)

    @pl.when(k == 0)
    def _():
        o_ref[...] = jnp.zeros_like(o_ref)

    o_ref[...] += lax.dot_general(
        lat_ref[...], wd_ref[...],
        dimension_numbers=(((1,), (1,)), ((), ())),
        preferred_element_type=jnp.float32,
    )


def _decoder(latents, Wd):
    return pl.pallas_call(
        _dec_body,
        grid=(NB_DEC,),
        in_specs=[
            pl.BlockSpec((128, KBLK), lambda k: (0, k)),
            pl.BlockSpec((DM, KBLK), lambda k: (0, k)),
        ],
        out_specs=pl.BlockSpec((128, DM), lambda k: (0, 0)),
        out_shape=jax.ShapeDtypeStruct((128, DM), jnp.float32),
    )(latents, Wd)


def kernel(x, We, be, Wd):
    pre, bmax, tau = _encoder(x, We, be)
    latents = _sc_topk_latents(pre, bmax, tau)
    recon = _decoder(latents, Wd)
    return (recon, latents, pre)
